# baseline (speedup 1.0000x reference)
import jax
import jax.numpy as jnp
from jax import lax
from jax.experimental import pallas as pl
from jax.experimental.pallas import tpu as pltpu
from jax.experimental.pallas import tpu_sc as plsc

B, C, H, W = 16, 384, 56, 56
P = B * H * W
NW = 32
PPW = P // NW
GP = 32
NG = PPW // GP
NSLOT = 2
CH16 = C // 16
UNROLL = 16


def _exchange_body(x0_hbm, x1_hbm, w0_hbm, w1_hbm, o1_hbm, o2_hbm,
                   bufs, vecs, sems):
    bx0 = bufs[0:NSLOT]
    bx1 = bufs[NSLOT:2 * NSLOT]
    bo1 = bufs[2 * NSLOT:3 * NSLOT]
    bo2 = bufs[3 * NSLOT:4 * NSLOT]
    ns2 = 2 * NSLOT
    sx0 = sems[0:ns2]
    sx1 = sems[ns2:2 * ns2]
    so1 = sems[2 * ns2:3 * ns2]
    so2 = sems[3 * ns2:4 * ns2]
    w0v, w1v, m1v, m2v = vecs

    wid = lax.axis_index("c") * 16 + lax.axis_index("s")
    base_w = wid * PPW

    HGP = GP // 2

    def in_copies(g, s):
        base = base_w + g * GP
        cs = []
        for hbm, buf, sem in ((x0_hbm, bx0[s], sx0), (x1_hbm, bx1[s], sx1)):
            for h, off in enumerate((0, HGP)):
                cs.append(pltpu.make_async_copy(
                    hbm.at[pl.ds(base + off, HGP)],
                    buf.at[pl.ds(off, HGP)], sem[2 * s + h]))
        return cs

    def out_copies(g, s):
        base = base_w + g * GP
        cs = []
        for hbm, buf, sem in ((o1_hbm, bo1[s], so1), (o2_hbm, bo2[s], so2)):
            for h, off in enumerate((0, HGP)):
                cs.append(pltpu.make_async_copy(
                    buf.at[pl.ds(off, HGP)],
                    hbm.at[pl.ds(base + off, HGP)], sem[2 * s + h]))
        return cs

    for s in range(NSLOT):
        for c in in_copies(s, s):
            c.start()

    pltpu.sync_copy(w0_hbm, w0v)
    pltpu.sync_copy(w1_hbm, w1v)

    mn = jnp.abs(w0v[pl.ds(0, 16)])
    mx = mn
    for j in range(1, CH16):
        a = jnp.abs(w0v[pl.ds(j * 16, 16)])
        mn = jnp.minimum(mn, a)
        mx = jnp.maximum(mx, a)
    dnums = lax.GatherDimensionNumbers(
        offset_dims=(), collapsed_slice_dims=(0,), start_index_map=(0,))

    def shuffle(v, perm):
        return lax.gather(v, perm[:, None], dnums, slice_sizes=(1,),
                          mode=lax.GatherScatterMode.PROMISE_IN_BOUNDS)

    idx = lax.iota(jnp.int32, 16)
    for k in (8, 4, 2, 1):
        perm = jnp.bitwise_xor(idx, k)
        mn = jnp.minimum(mn, shuffle(mn, perm))
        mx = jnp.maximum(mx, shuffle(mx, perm))
    thrv = mn + 0.05 * (mx - mn)

    one = jnp.full((16,), 1.0, jnp.float32)
    zero = jnp.full((16,), 0.0, jnp.float32)
    for j in range(CH16):
        sl = pl.ds(j * 16, 16)
        m1v[sl] = jnp.where(jnp.abs(w0v[sl]) >= thrv, one, zero)
        m2v[sl] = jnp.where(jnp.abs(w1v[sl]) >= thrv, one, zero)

    def compute(s):
        bi0, bi1, bu1, bu2 = bx0[s], bx1[s], bo1[s], bo2[s]
        for j in range(CH16):
            sl = pl.ds(j * 16, 16)
            m1 = m1v[sl] > zero
            m2 = m2v[sl] > zero

            @plsc.parallel_loop(0, GP, 1, unroll=UNROLL)
            def _(p):
                a = bi0[p, sl]
                b = bi1[p, sl]
                prod = a * b
                bu1[p, sl] = jnp.where(m1, a, prod)
                bu2[p, sl] = jnp.where(m2, b, prod)

    def step(g, s, first_round):
        for c in in_copies(g, s):
            c.wait()
        if first_round is None:
            for c in out_copies(g - NSLOT, s):
                c.wait()
        else:
            @pl.when(jnp.logical_not(first_round))
            def _():
                for c in out_copies(g - NSLOT, s):
                    c.wait()

        compute(s)
        for c in out_copies(g, s):
            c.start()

        @pl.when(g + NSLOT < NG)
        def _():
            for c in in_copies(g + NSLOT, s):
                c.start()

    def group(t, _):
        for s in range(NSLOT):
            step(t * NSLOT + s, s, t == 0)
        return 0

    lax.fori_loop(0, NG // NSLOT, group, 0)

    for e in range(NG - NG % NSLOT, NG):
        step(e, e % NSLOT, None)

    for g in range(NG - NSLOT, NG):
        for c in out_copies(g, g % NSLOT):
            c.wait()


def kernel(x0, x1, insnorm_weight0, insnorm_weight1, threshold):
    del threshold
    x0t = jnp.transpose(x0, (0, 2, 3, 1)).reshape(P, C)
    x1t = jnp.transpose(x1, (0, 2, 3, 1)).reshape(P, C)

    f32 = jnp.float32
    buf = pltpu.VMEM((GP, C), f32)
    vec = pltpu.VMEM((C,), f32)
    sem = pltpu.SemaphoreType.DMA

    def body(x0r, x1r, w0r, w1r, o1r, o2r, *scratch):
        nb = 4 * NSLOT
        _exchange_body(x0r, x1r, w0r, w1r, o1r, o2r,
                       scratch[:nb], scratch[nb:nb + 4], scratch[nb + 4:])

    run = pl.kernel(
        body,
        out_type=(
            jax.ShapeDtypeStruct((P, C), f32),
            jax.ShapeDtypeStruct((P, C), f32),
        ),
        mesh=plsc.VectorSubcoreMesh(core_axis_name="c", subcore_axis_name="s"),
        scratch_types=(buf,) * (4 * NSLOT) + (vec,) * 4 + (sem,) * (8 * NSLOT),
    )
    o1, o2 = run(x0t, x1t, insnorm_weight0, insnorm_weight1)
    o1 = jnp.transpose(o1.reshape(B, H, W, C), (0, 3, 1, 2))
    o2 = jnp.transpose(o2.reshape(B, H, W, C), (0, 3, 1, 2))
    return (o1, o2)


# speedup vs baseline: 1.1641x; 1.1641x over previous
import jax
import jax.numpy as jnp
from jax import lax
from jax.experimental import pallas as pl
from jax.experimental.pallas import tpu as pltpu
from jax.experimental.pallas import tpu_sc as plsc

B, C, H, W = 16, 384, 56, 56
P = B * H * W
NW = 32
PPW = P // NW
GP = 16
NG = PPW // GP
NSLOT = 2
CH16 = C // 16
UNROLL = 8


def _exchange_body(x0_hbm, x1_hbm, w0_hbm, w1_hbm, o1_hbm, o2_hbm,
                   bufs, vecs, sems):
    bx0 = bufs[0:NSLOT]
    bx1 = bufs[NSLOT:2 * NSLOT]
    bo1 = bufs[2 * NSLOT:3 * NSLOT]
    bo2 = bufs[3 * NSLOT:4 * NSLOT]
    sx0 = sems[0:NSLOT]
    sx1 = sems[NSLOT:2 * NSLOT]
    so1 = sems[2 * NSLOT:3 * NSLOT]
    so2 = sems[3 * NSLOT:4 * NSLOT]
    w0v, w1v, m1v, m2v = vecs

    wid = lax.axis_index("c") * 16 + lax.axis_index("s")
    base_w = wid * PPW

    def in_copies(g, s):
        base = base_w + g * GP
        return (
            pltpu.make_async_copy(x0_hbm.at[pl.ds(base, GP)], bx0[s], sx0[s]),
            pltpu.make_async_copy(x1_hbm.at[pl.ds(base, GP)], bx1[s], sx1[s]),
        )

    def out_copies(g, s):
        base = base_w + g * GP
        return (
            pltpu.make_async_copy(bo1[s], o1_hbm.at[pl.ds(base, GP)], so1[s]),
            pltpu.make_async_copy(bo2[s], o2_hbm.at[pl.ds(base, GP)], so2[s]),
        )

    for s in range(NSLOT):
        for c in in_copies(s, s):
            c.start()

    pltpu.sync_copy(w0_hbm, w0v)
    pltpu.sync_copy(w1_hbm, w1v)

    mn = jnp.abs(w0v[pl.ds(0, 16)])
    mx = mn
    for j in range(1, CH16):
        a = jnp.abs(w0v[pl.ds(j * 16, 16)])
        mn = jnp.minimum(mn, a)
        mx = jnp.maximum(mx, a)
    dnums = lax.GatherDimensionNumbers(
        offset_dims=(), collapsed_slice_dims=(0,), start_index_map=(0,))

    def shuffle(v, perm):
        return lax.gather(v, perm[:, None], dnums, slice_sizes=(1,),
                          mode=lax.GatherScatterMode.PROMISE_IN_BOUNDS)

    idx = lax.iota(jnp.int32, 16)
    for k in (8, 4, 2, 1):
        perm = jnp.bitwise_xor(idx, k)
        mn = jnp.minimum(mn, shuffle(mn, perm))
        mx = jnp.maximum(mx, shuffle(mx, perm))
    thrv = mn + 0.05 * (mx - mn)

    one = jnp.full((16,), 1.0, jnp.float32)
    zero = jnp.full((16,), 0.0, jnp.float32)
    for j in range(CH16):
        sl = pl.ds(j * 16, 16)
        m1v[sl] = jnp.where(jnp.abs(w0v[sl]) >= thrv, one, zero)
        m2v[sl] = jnp.where(jnp.abs(w1v[sl]) >= thrv, one, zero)

    def compute(s):
        bi0, bi1, bu1, bu2 = bx0[s], bx1[s], bo1[s], bo2[s]
        for j in range(CH16):
            sl = pl.ds(j * 16, 16)
            m1 = m1v[sl] > zero
            m2 = m2v[sl] > zero

            @plsc.parallel_loop(0, GP, 1, unroll=UNROLL)
            def _(p):
                a = bi0[p, sl]
                b = bi1[p, sl]
                prod = a * b
                bu1[p, sl] = jnp.where(m1, a, prod)
                bu2[p, sl] = jnp.where(m2, b, prod)

    def step(g, s, first_round):
        for c in in_copies(g, s):
            c.wait()
        if first_round is None:
            for c in out_copies(g - NSLOT, s):
                c.wait()
        else:
            @pl.when(jnp.logical_not(first_round))
            def _():
                for c in out_copies(g - NSLOT, s):
                    c.wait()

        compute(s)
        for c in out_copies(g, s):
            c.start()

        @pl.when(g + NSLOT < NG)
        def _():
            for c in in_copies(g + NSLOT, s):
                c.start()

    def group(t, _):
        for s in range(NSLOT):
            step(t * NSLOT + s, s, t == 0)
        return 0

    lax.fori_loop(0, NG // NSLOT, group, 0)

    for e in range(NG - NG % NSLOT, NG):
        step(e, e % NSLOT, None)

    for g in range(NG - NSLOT, NG):
        for c in out_copies(g, g % NSLOT):
            c.wait()


def kernel(x0, x1, insnorm_weight0, insnorm_weight1, threshold):
    del threshold
    x0t = jnp.transpose(x0, (0, 2, 3, 1)).reshape(P, C)
    x1t = jnp.transpose(x1, (0, 2, 3, 1)).reshape(P, C)

    f32 = jnp.float32
    buf = pltpu.VMEM((GP, C), f32)
    vec = pltpu.VMEM((C,), f32)
    sem = pltpu.SemaphoreType.DMA

    def body(x0r, x1r, w0r, w1r, o1r, o2r, *scratch):
        nb = 4 * NSLOT
        _exchange_body(x0r, x1r, w0r, w1r, o1r, o2r,
                       scratch[:nb], scratch[nb:nb + 4], scratch[nb + 4:])

    run = pl.kernel(
        body,
        out_type=(
            jax.ShapeDtypeStruct((P, C), f32),
            jax.ShapeDtypeStruct((P, C), f32),
        ),
        mesh=plsc.VectorSubcoreMesh(core_axis_name="c", subcore_axis_name="s"),
        scratch_types=(buf,) * (4 * NSLOT) + (vec,) * 4 + (sem,) * (4 * NSLOT),
    )
    o1, o2 = run(x0t, x1t, insnorm_weight0, insnorm_weight1)
    o1 = jnp.transpose(o1.reshape(B, H, W, C), (0, 3, 1, 2))
    o2 = jnp.transpose(o2.reshape(B, H, W, C), (0, 3, 1, 2))
    return (o1, o2)
